# baseline (speedup 1.0000x reference)
import functools

import jax
import jax.numpy as jnp
from jax import lax
from jax.experimental import pallas as pl
from jax.experimental.pallas import tpu as pltpu
from jax.experimental.pallas import tpu_sc as plsc

N_TASKS = 8
N_EXP = 64
TOPK = 2
WMI = 0.01

NC = 2
NS = 16
NW = NC * NS
LANES = 16
TOKENS = 32768
ROWS_PER_W = TOKENS // NW
CHUNK = 128
KV = N_EXP // LANES

_mesh = plsc.VectorSubcoreMesh(core_axis_name="c", subcore_axis_name="s")


@functools.partial(
    pl.kernel,
    mesh=_mesh,
    out_type=jax.ShapeDtypeStruct((NW, N_TASKS, N_EXP), jnp.float32),
    scratch_types=[
        pltpu.VMEM((2, CHUNK, N_EXP), jnp.float32),
        pltpu.VMEM((ROWS_PER_W + LANES,), jnp.int32),
        pltpu.VMEM((N_TASKS, N_EXP), jnp.float32),
        pltpu.SemaphoreType.DMA((2,)),
        pltpu.SemaphoreType.DMA,
    ],
)
def _sc_partial(logits_hbm, labels_hbm, out_hbm, lbuf, labbuf, acc,
                lsem, labsem):
    wid = lax.axis_index("s") * NC + lax.axis_index("c")
    base = wid * ROWS_PER_W
    n_chunks = ROWS_PER_W // CHUNK

    lab_cp = pltpu.async_copy(labels_hbm.at[pl.ds(base, ROWS_PER_W)],
                              labbuf.at[pl.ds(0, ROWS_PER_W)], labsem)

    def start_chunk(g):
        return pltpu.async_copy(
            logits_hbm.at[pl.ds(base + g * CHUNK, CHUNK)], lbuf.at[g % 2],
            lsem.at[g % 2])

    copies = [start_chunk(0)]

    zeros = jnp.zeros((LANES,), jnp.float32)
    for t in range(N_TASKS):
        for k in range(KV):
            acc[t, pl.ds(k * LANES, LANES)] = zeros

    perms = [lax.iota(jnp.int32, LANES) ^ s for s in (8, 4, 2, 1)]

    lab_cp.wait()
    for g in range(n_chunks):
        copies[g].wait()
        if g + 1 < n_chunks:
            copies.append(start_chunk(g + 1))
        buf = lbuf.at[g % 2]

        @plsc.parallel_loop(0, CHUNK, unroll=2)
        def sm_row(r):
            vs = [buf[r, pl.ds(k * LANES, LANES)] for k in range(KV)]
            m = jnp.maximum(jnp.maximum(vs[0], vs[1]),
                            jnp.maximum(vs[2], vs[3]))
            for p in perms:
                m = jnp.maximum(m, m[p])
            es = [jnp.exp(v - m) for v in vs]
            s = (es[0] + es[1]) + (es[2] + es[3])
            for p in perms:
                s = s + s[p]
            inv = 1.0 / s
            lab = labbuf[pl.ds(g * CHUNK + r, LANES)][0]
            for k in range(KV):
                plsc.addupdate(acc.at[lab, pl.ds(k * LANES, LANES)],
                               es[k] * inv)

    pltpu.sync_copy(acc, out_hbm.at[wid])


def _fin_body(p_ref, o_ref):
    S = jnp.sum(p_ref[...], axis=0)
    counts = jnp.sum(S, axis=1, keepdims=True)
    MI = counts * S
    tot = jnp.sum(MI) / TOPK
    MI = MI / (tot + 0.0001)
    P_TI = jnp.sum(MI, axis=1, keepdims=True) + 0.0001
    P_EI = jnp.sum(MI, axis=0, keepdims=True) + 0.0001
    loss = -jnp.sum(MI * jnp.log(MI / P_TI / P_EI + 0.0001))
    o_ref[...] = jnp.reshape(WMI * loss, (1, 1))


_finalize = pl.pallas_call(
    _fin_body,
    out_shape=jax.ShapeDtypeStruct((1, 1), jnp.float32),
)


def kernel(router_logits, router_labels):
    labels = router_labels.reshape(-1).astype(jnp.int32)
    partials = _sc_partial(router_logits, labels)
    return _finalize(partials)[0, 0]


# speedup vs baseline: 1.0468x; 1.0468x over previous
import functools

import jax
import jax.numpy as jnp
from jax import lax
from jax.experimental import pallas as pl
from jax.experimental.pallas import tpu as pltpu
from jax.experimental.pallas import tpu_sc as plsc

N_TASKS = 8
N_EXP = 64
TOPK = 2
WMI = 0.01

NC = 2
NS = 16
NW = NC * NS
LANES = 16
TOKENS = 32768
ROWS_PER_W = TOKENS // NW
CHUNK = 256
KV = N_EXP // LANES

_mesh = plsc.VectorSubcoreMesh(core_axis_name="c", subcore_axis_name="s")


@functools.partial(
    pl.kernel,
    mesh=_mesh,
    out_type=jax.ShapeDtypeStruct((NW, N_TASKS, N_EXP), jnp.float32),
    scratch_types=[
        pltpu.VMEM((2, CHUNK, N_EXP), jnp.float32),
        pltpu.VMEM((ROWS_PER_W + LANES,), jnp.int32),
        pltpu.VMEM((N_TASKS, N_EXP), jnp.float32),
        pltpu.SemaphoreType.DMA((2,)),
        pltpu.SemaphoreType.DMA,
    ],
)
def _sc_partial(logits_hbm, labels_hbm, out_hbm, lbuf, labbuf, acc,
                lsem, labsem):
    wid = lax.axis_index("s") * NC + lax.axis_index("c")
    base = wid * ROWS_PER_W
    n_chunks = ROWS_PER_W // CHUNK

    lab_cp = pltpu.async_copy(labels_hbm.at[pl.ds(base, ROWS_PER_W)],
                              labbuf.at[pl.ds(0, ROWS_PER_W)], labsem)

    def start_chunk(g):
        return pltpu.async_copy(
            logits_hbm.at[pl.ds(base + g * CHUNK, CHUNK)], lbuf.at[g % 2],
            lsem.at[g % 2])

    copies = [start_chunk(0)]

    zeros = jnp.zeros((LANES,), jnp.float32)
    for t in range(N_TASKS):
        for k in range(KV):
            acc[t, pl.ds(k * LANES, LANES)] = zeros

    perms = [lax.iota(jnp.int32, LANES) ^ s for s in (8, 4, 2, 1)]

    lab_cp.wait()
    for g in range(n_chunks):
        copies[g].wait()
        if g + 1 < n_chunks:
            copies.append(start_chunk(g + 1))
        buf = lbuf.at[g % 2]

        @plsc.parallel_loop(0, CHUNK)
        def sm_row(r):
            vs = [buf[r, pl.ds(k * LANES, LANES)] for k in range(KV)]
            m = jnp.maximum(jnp.maximum(vs[0], vs[1]),
                            jnp.maximum(vs[2], vs[3]))
            for p in perms:
                m = jnp.maximum(m, m[p])
            es = [jnp.exp(v - m) for v in vs]
            s = (es[0] + es[1]) + (es[2] + es[3])
            for p in perms:
                s = s + s[p]
            inv = 1.0 / s
            lab = labbuf[pl.ds(g * CHUNK + r, LANES)][0]
            for k in range(KV):
                plsc.addupdate(acc.at[lab, pl.ds(k * LANES, LANES)],
                               es[k] * inv)

    pltpu.sync_copy(acc, out_hbm.at[wid])


def _fin_body(p_ref, o_ref):
    S = jnp.sum(p_ref[...], axis=0)
    counts = jnp.sum(S, axis=1, keepdims=True)
    MI = counts * S
    tot = jnp.sum(MI) / TOPK
    MI = MI / (tot + 0.0001)
    P_TI = jnp.sum(MI, axis=1, keepdims=True) + 0.0001
    P_EI = jnp.sum(MI, axis=0, keepdims=True) + 0.0001
    loss = -jnp.sum(MI * jnp.log(MI / P_TI / P_EI + 0.0001))
    o_ref[...] = jnp.reshape(WMI * loss, (1, 1))


_finalize = pl.pallas_call(
    _fin_body,
    out_shape=jax.ShapeDtypeStruct((1, 1), jnp.float32),
)


def kernel(router_logits, router_labels):
    labels = router_labels.reshape(-1).astype(jnp.int32)
    partials = _sc_partial(router_logits, labels)
    return _finalize(partials)[0, 0]
